# baseline (speedup 1.0000x reference)
import functools

import jax
import jax.numpy as jnp
from jax import lax
from jax.experimental import pallas as pl
from jax.experimental.pallas import tpu as pltpu
from jax.experimental.pallas import tpu_sc as plsc

N = 10000
E = 320000
D = 128
H = 128
OUT = 10
G = 64

NC = 2
NS = 16
NW = NC * NS
EP = E // NW
KE = 80
NCHUNK = EP // KE
KE2 = 128
EPP = 10240
NCH = EPP // KE2
NPAIR = NCH // 2
NPAD = 10240
RPS = NPAD // NS
WCH = 128
NWC = RPS // WCH

_mesh = plsc.VectorSubcoreMesh(
    core_axis_name="c", subcore_axis_name="s", num_cores=NC, num_subcores=NS)


def _edge_body(hw_hbm, src_hbm, dst_hbm, z_hbm, out_hbm,
               isrc0, idst0, rows0, isrc1, idst1, rows1,
               acc_sh, gsem0, gsem1, isem0, isem1):
    cid = lax.axis_index("c")
    sid = lax.axis_index("s")
    wid = cid * NS + sid
    pltpu.sync_copy(z_hbm, rows0)

    def zero_chunk(j, carry):
        pltpu.sync_copy(rows0, acc_sh.at[pl.ds(sid * RPS + j * WCH, WCH)])
        return carry

    lax.fori_loop(0, NWC, zero_chunk, 0)
    plsc.subcore_barrier()

    ebase = wid * EPP
    bufs = ((isrc0, idst0, rows0, gsem0, isem0),
            (isrc1, idst1, rows1, gsem1, isem1))

    def stage(b, c):
        isrc, idst, rows, gsem, isem = bufs[b]
        pltpu.async_copy(src_hbm.at[pl.ds(ebase + c * KE2, KE2)], isrc, isem)
        pltpu.async_copy(dst_hbm.at[pl.ds(ebase + c * KE2, KE2)], idst, isem)

    def launch(b):
        isrc, idst, rows, gsem, isem = bufs[b]
        pltpu.make_async_copy(src_hbm.at[pl.ds(0, KE2)], isrc, isem).wait()
        pltpu.make_async_copy(dst_hbm.at[pl.ds(0, KE2)], idst, isem).wait()
        pltpu.async_copy(hw_hbm.at[isrc], rows, gsem)

    def consume(b):
        isrc, idst, rows, gsem, isem = bufs[b]
        pltpu.make_async_copy(hw_hbm.at[isrc], rows, gsem).wait()
        pltpu.sync_copy(rows, acc_sh.at[idst], add=True)

    stage(0, 0)
    launch(0)
    stage(1, 1)

    def pair(p, carry):
        launch(1)
        consume(0)
        stage(0, 2 * p + 2)
        launch(0)
        consume(1)
        stage(1, 2 * p + 3)
        return carry

    lax.fori_loop(0, NPAIR - 1, pair, 0)
    launch(1)
    consume(0)
    consume(1)
    plsc.subcore_barrier()

    def wb_chunk(j, carry):
        base = sid * RPS + j * WCH
        pltpu.sync_copy(acc_sh.at[pl.ds(base, WCH)], rows0)
        pltpu.sync_copy(rows0, out_hbm.at[cid, pl.ds(base, WCH)])
        return carry

    lax.fori_loop(0, NWC, wb_chunk, 0)


_edge_call = functools.partial(
    pl.kernel,
    out_type=jax.ShapeDtypeStruct((NC, NPAD, H), jnp.float32),
    mesh=_mesh,
    scratch_types=[
        pltpu.VMEM((KE2,), jnp.int32),
        pltpu.VMEM((KE2,), jnp.int32),
        pltpu.VMEM((KE2, H), jnp.float32),
        pltpu.VMEM((KE2,), jnp.int32),
        pltpu.VMEM((KE2,), jnp.int32),
        pltpu.VMEM((KE2, H), jnp.float32),
        pltpu.VMEM_SHARED((NPAD, H), jnp.float32),
        pltpu.SemaphoreType.DMA,
        pltpu.SemaphoreType.DMA,
        pltpu.SemaphoreType.DMA,
        pltpu.SemaphoreType.DMA,
    ],
)(_edge_body)


def _deg_body(dst_hbm, ones_hbm, z_hbm, out_hbm,
              idst_v, rows_v, stage_v, acc_sh, sem):
    cid = lax.axis_index("c")
    sid = lax.axis_index("s")
    wid = cid * NS + sid
    pltpu.sync_copy(z_hbm, stage_v)

    def zero_chunk(j, carry):
        pltpu.sync_copy(stage_v, acc_sh.at[pl.ds(sid * RPS + j * WCH, WCH)])
        return carry

    lax.fori_loop(0, NWC, zero_chunk, 0)
    pltpu.sync_copy(ones_hbm, rows_v)
    plsc.subcore_barrier()

    def chunk(i, carry):
        base = wid * EP + i * KE
        pltpu.sync_copy(dst_hbm.at[pl.ds(base, KE)], idst_v)
        pltpu.sync_copy(rows_v, acc_sh.at[idst_v], add=True)
        return carry

    lax.fori_loop(0, NCHUNK, chunk, 0)
    plsc.subcore_barrier()

    def wb_chunk(j, carry):
        base = sid * RPS + j * WCH
        pltpu.sync_copy(acc_sh.at[pl.ds(base, WCH)], stage_v)
        pltpu.sync_copy(stage_v, out_hbm.at[cid, pl.ds(base, WCH)])
        return carry

    lax.fori_loop(0, NWC, wb_chunk, 0)


_deg_call = functools.partial(
    pl.kernel,
    out_type=jax.ShapeDtypeStruct((NC, NPAD, H), jnp.float32),
    mesh=_mesh,
    scratch_types=[
        pltpu.VMEM((KE,), jnp.int32),
        pltpu.VMEM((KE, H), jnp.float32),
        pltpu.VMEM((WCH, H), jnp.float32),
        pltpu.VMEM_SHARED((NPAD, H), jnp.float32),
        pltpu.SemaphoreType.DMA,
    ],
)(_deg_body)


def _prep_body(degp_ref, x_ref, w0_ref, dinv_ref, hw0_ref):
    deg = degp_ref[0][0:N, 0:1] + degp_ref[1][0:N, 0:1] + 1.0
    dinv = 1.0 / jnp.sqrt(deg)
    dinv_ref[...] = dinv
    hw0_ref[...] = dinv * jnp.dot(x_ref[...], w0_ref[...],
                                  preferred_element_type=jnp.float32)


def _bn_block(accp_ref, hw_ref, dinv_ref, b_ref, g_ref, be_ref):
    dinv = dinv_ref[...]
    pre = dinv * (accp_ref[0][0:N] + accp_ref[1][0:N] + hw_ref[...]) + b_ref[...]
    r = jnp.maximum(pre, 0.0)
    mean = jnp.mean(r, axis=0, keepdims=True)
    var = jnp.mean((r - mean) ** 2, axis=0, keepdims=True)
    return (r - mean) / jnp.sqrt(var + 1e-5) * g_ref[...] + be_ref[...]


def _mid_body(accp_ref, hw_ref, dinv_ref, b_ref, g_ref, be_ref, wn_ref, hwn_ref):
    hb = _bn_block(accp_ref, hw_ref, dinv_ref, b_ref, g_ref, be_ref)
    hwn_ref[...] = dinv_ref[...] * jnp.dot(hb, wn_ref[...],
                                           preferred_element_type=jnp.float32)


def _final_body(accp_ref, hw_ref, dinv_ref, b_ref, g_ref, be_ref, batch_ref,
                wh1_ref, bh1_ref, wh2_ref, bh2_ref, wo_ref, bo_ref, out_ref):
    h = _bn_block(accp_ref, hw_ref, dinv_ref, b_ref, g_ref, be_ref)
    onehot = (batch_ref[...] == lax.broadcasted_iota(jnp.int32, (1, G), 1)
              ).astype(jnp.float32)
    sums = lax.dot_general(onehot, h, (((0,), (0,)), ((), ())),
                           preferred_element_type=jnp.float32)
    counts = lax.dot_general(onehot, jnp.ones((N, 1), jnp.float32),
                             (((0,), (0,)), ((), ())),
                             preferred_element_type=jnp.float32)
    pooled = sums / jnp.maximum(counts, 1.0)
    z = jax.nn.gelu(pooled @ wh1_ref[...] + bh1_ref[...])
    z = jax.nn.gelu(z @ wh2_ref[...] + bh2_ref[...])
    out_ref[...] = z @ wo_ref[...] + bo_ref[...]


def kernel(x, edge_index, batch, W0, b0, g0, be0, W1, b1, g1, be1,
           W2, b2, g2, be2, Wh1, bh1, Wh2, bh2, Wo, bo):
    src = edge_index[0]
    dst = edge_index[1]
    pad = jnp.zeros((NW, EPP - EP), jnp.int32)
    srcp = jnp.concatenate([src.reshape(NW, EP), pad], axis=1).reshape(-1)
    dstp = jnp.concatenate([dst.reshape(NW, EP), pad + (NPAD - 1)],
                           axis=1).reshape(-1)
    z128 = jnp.zeros((WCH, H), jnp.float32)
    ones128 = jnp.ones((KE, H), jnp.float32)

    degp = _deg_call(dst, ones128, z128)

    dinv, hw = pl.pallas_call(
        _prep_body,
        out_shape=(jax.ShapeDtypeStruct((N, 1), jnp.float32),
                   jax.ShapeDtypeStruct((N, H), jnp.float32)),
    )(degp, x, W0)

    for (bi, gi, bei, Wn) in ((b0, g0, be0, W1), (b1, g1, be1, W2)):
        accp = _edge_call(hw, srcp, dstp, z128)
        hw = pl.pallas_call(
            _mid_body,
            out_shape=jax.ShapeDtypeStruct((N, H), jnp.float32),
        )(accp, hw, dinv, bi.reshape(1, H), gi.reshape(1, H),
          bei.reshape(1, H), Wn)

    accp = _edge_call(hw, srcp, dstp, z128)
    out = pl.pallas_call(
        _final_body,
        out_shape=jax.ShapeDtypeStruct((G, OUT), jnp.float32),
    )(accp, hw, dinv, b2.reshape(1, H), g2.reshape(1, H), be2.reshape(1, H),
      batch.reshape(N, 1), Wh1, bh1.reshape(1, H), Wh2, bh2.reshape(1, H),
      Wo, bo.reshape(1, OUT))
    return out


# speedup vs baseline: 1.4643x; 1.4643x over previous
import functools

import jax
import jax.numpy as jnp
from jax import lax
from jax.experimental import pallas as pl
from jax.experimental.pallas import tpu as pltpu
from jax.experimental.pallas import tpu_sc as plsc

N = 10000
E = 320000
D = 128
H = 128
OUT = 10
G = 64

NC = 2
NS = 16
NW = NC * NS
EP = E // NW
KE = 80
NCHUNK = EP // KE
NPAD = 10240
RPS = NPAD // NS
WCH = 128
NWC = RPS // WCH

_mesh = plsc.VectorSubcoreMesh(
    core_axis_name="c", subcore_axis_name="s", num_cores=NC, num_subcores=NS)


def _edge_body(hw_hbm, src_hbm, dst_hbm, z_hbm, out_hbm,
               isrc0, idst0, rows0, isrc1, idst1, rows1,
               stage_v, acc_sh, gsem0, gsem1):
    cid = lax.axis_index("c")
    sid = lax.axis_index("s")
    wid = cid * NS + sid
    pltpu.sync_copy(z_hbm, stage_v)

    def zero_chunk(j, carry):
        pltpu.sync_copy(stage_v, acc_sh.at[pl.ds(sid * RPS + j * WCH, WCH)])
        return carry

    lax.fori_loop(0, NWC, zero_chunk, 0)
    plsc.subcore_barrier()

    ebase = wid * EP
    bufs = ((isrc0, idst0, rows0, gsem0),
            (isrc1, idst1, rows1, gsem1))

    def stage(buf, base):
        isrc, idst, rows, gsem = buf
        pltpu.sync_copy(src_hbm.at[pl.ds(base, KE)], isrc)
        pltpu.sync_copy(dst_hbm.at[pl.ds(base, KE)], idst)
        pltpu.async_copy(hw_hbm.at[isrc], rows, gsem)

    stage(bufs[0], ebase)

    def pair(p, carry):
        isrc_a, idst_a, rows_a, gsem_a = bufs[0]
        isrc_b, idst_b, rows_b, gsem_b = bufs[1]
        pltpu.make_async_copy(hw_hbm.at[isrc_a], rows_a, gsem_a).wait()
        stage(bufs[1], ebase + (2 * p + 1) * KE)
        pltpu.sync_copy(rows_a, acc_sh.at[idst_a], add=True)
        pltpu.make_async_copy(hw_hbm.at[isrc_b], rows_b, gsem_b).wait()
        stage(bufs[0], ebase + (2 * p + 2) * KE)
        pltpu.sync_copy(rows_b, acc_sh.at[idst_b], add=True)
        return carry

    lax.fori_loop(0, NCHUNK // 2, pair, 0)
    isrc_t, idst_t, rows_t, gsem_t = bufs[0]
    pltpu.make_async_copy(hw_hbm.at[isrc_t], rows_t, gsem_t).wait()
    pltpu.sync_copy(rows_t, acc_sh.at[idst_t], add=True)
    plsc.subcore_barrier()

    def wb_chunk(j, carry):
        base = sid * RPS + j * WCH
        pltpu.sync_copy(acc_sh.at[pl.ds(base, WCH)], stage_v)
        pltpu.sync_copy(stage_v, out_hbm.at[cid, pl.ds(base, WCH)])
        return carry

    lax.fori_loop(0, NWC, wb_chunk, 0)


_edge_call = functools.partial(
    pl.kernel,
    out_type=jax.ShapeDtypeStruct((NC, NPAD, H), jnp.float32),
    mesh=_mesh,
    scratch_types=[
        pltpu.VMEM((KE,), jnp.int32),
        pltpu.VMEM((KE,), jnp.int32),
        pltpu.VMEM((KE, H), jnp.float32),
        pltpu.VMEM((KE,), jnp.int32),
        pltpu.VMEM((KE,), jnp.int32),
        pltpu.VMEM((KE, H), jnp.float32),
        pltpu.VMEM((WCH, H), jnp.float32),
        pltpu.VMEM_SHARED((NPAD, H), jnp.float32),
        pltpu.SemaphoreType.DMA,
        pltpu.SemaphoreType.DMA,
    ],
)(_edge_body)


def _deg_body(dst_hbm, ones_hbm, z_hbm, out_hbm,
              idst_v, rows_v, stage_v, acc_sh, sem):
    cid = lax.axis_index("c")
    sid = lax.axis_index("s")
    wid = cid * NS + sid
    pltpu.sync_copy(z_hbm, stage_v)

    def zero_chunk(j, carry):
        pltpu.sync_copy(stage_v, acc_sh.at[pl.ds(sid * RPS + j * WCH, WCH)])
        return carry

    lax.fori_loop(0, NWC, zero_chunk, 0)
    pltpu.sync_copy(ones_hbm, rows_v)
    plsc.subcore_barrier()

    def chunk(i, carry):
        base = wid * EP + i * KE
        pltpu.sync_copy(dst_hbm.at[pl.ds(base, KE)], idst_v)
        pltpu.sync_copy(rows_v, acc_sh.at[idst_v], add=True)
        return carry

    lax.fori_loop(0, NCHUNK, chunk, 0)
    plsc.subcore_barrier()

    def wb_chunk(j, carry):
        base = sid * RPS + j * WCH
        pltpu.sync_copy(acc_sh.at[pl.ds(base, WCH)], stage_v)
        pltpu.sync_copy(stage_v, out_hbm.at[cid, pl.ds(base, WCH)])
        return carry

    lax.fori_loop(0, NWC, wb_chunk, 0)


_deg_call = functools.partial(
    pl.kernel,
    out_type=jax.ShapeDtypeStruct((NC, NPAD, H), jnp.float32),
    mesh=_mesh,
    scratch_types=[
        pltpu.VMEM((KE,), jnp.int32),
        pltpu.VMEM((KE, H), jnp.float32),
        pltpu.VMEM((WCH, H), jnp.float32),
        pltpu.VMEM_SHARED((NPAD, H), jnp.float32),
        pltpu.SemaphoreType.DMA,
    ],
)(_deg_body)


def _prep_body(degp_ref, x_ref, w0_ref, dinv_ref, hw0_ref):
    deg = degp_ref[0][0:N, 0:1] + degp_ref[1][0:N, 0:1] + 1.0
    dinv = 1.0 / jnp.sqrt(deg)
    dinv_ref[...] = dinv
    hw0_ref[...] = dinv * jnp.dot(x_ref[...], w0_ref[...],
                                  preferred_element_type=jnp.float32)


def _bn_block(accp_ref, hw_ref, dinv_ref, b_ref, g_ref, be_ref):
    dinv = dinv_ref[...]
    pre = dinv * (accp_ref[0][0:N] + accp_ref[1][0:N] + hw_ref[...]) + b_ref[...]
    r = jnp.maximum(pre, 0.0)
    mean = jnp.mean(r, axis=0, keepdims=True)
    var = jnp.mean((r - mean) ** 2, axis=0, keepdims=True)
    return (r - mean) / jnp.sqrt(var + 1e-5) * g_ref[...] + be_ref[...]


def _mid_body(accp_ref, hw_ref, dinv_ref, b_ref, g_ref, be_ref, wn_ref, hwn_ref):
    hb = _bn_block(accp_ref, hw_ref, dinv_ref, b_ref, g_ref, be_ref)
    hwn_ref[...] = dinv_ref[...] * jnp.dot(hb, wn_ref[...],
                                           preferred_element_type=jnp.float32)


def _final_body(accp_ref, hw_ref, dinv_ref, b_ref, g_ref, be_ref, batch_ref,
                wh1_ref, bh1_ref, wh2_ref, bh2_ref, wo_ref, bo_ref, out_ref):
    h = _bn_block(accp_ref, hw_ref, dinv_ref, b_ref, g_ref, be_ref)
    onehot = (batch_ref[...] == lax.broadcasted_iota(jnp.int32, (1, G), 1)
              ).astype(jnp.float32)
    sums = lax.dot_general(onehot, h, (((0,), (0,)), ((), ())),
                           preferred_element_type=jnp.float32)
    counts = lax.dot_general(onehot, jnp.ones((N, 1), jnp.float32),
                             (((0,), (0,)), ((), ())),
                             preferred_element_type=jnp.float32)
    pooled = sums / jnp.maximum(counts, 1.0)
    z = jax.nn.gelu(pooled @ wh1_ref[...] + bh1_ref[...])
    z = jax.nn.gelu(z @ wh2_ref[...] + bh2_ref[...])
    out_ref[...] = z @ wo_ref[...] + bo_ref[...]


def kernel(x, edge_index, batch, W0, b0, g0, be0, W1, b1, g1, be1,
           W2, b2, g2, be2, Wh1, bh1, Wh2, bh2, Wo, bo):
    src = edge_index[0]
    dst = edge_index[1]
    z128 = jnp.zeros((WCH, H), jnp.float32)
    ones128 = jnp.ones((KE, H), jnp.float32)

    degp = _deg_call(dst, ones128, z128)

    dinv, hw = pl.pallas_call(
        _prep_body,
        out_shape=(jax.ShapeDtypeStruct((N, 1), jnp.float32),
                   jax.ShapeDtypeStruct((N, H), jnp.float32)),
    )(degp, x, W0)

    for (bi, gi, bei, Wn) in ((b0, g0, be0, W1), (b1, g1, be1, W2)):
        accp = _edge_call(hw, src, dst, z128)
        hw = pl.pallas_call(
            _mid_body,
            out_shape=jax.ShapeDtypeStruct((N, H), jnp.float32),
        )(accp, hw, dinv, bi.reshape(1, H), gi.reshape(1, H),
          bei.reshape(1, H), Wn)

    accp = _edge_call(hw, src, dst, z128)
    out = pl.pallas_call(
        _final_body,
        out_shape=jax.ShapeDtypeStruct((G, OUT), jnp.float32),
    )(accp, hw, dinv, b2.reshape(1, H), g2.reshape(1, H), be2.reshape(1, H),
      batch.reshape(N, 1), Wh1, bh1.reshape(1, H), Wh2, bh2.reshape(1, H),
      Wo, bo.reshape(1, OUT))
    return out
